# Initial kernel scaffold; baseline (speedup 1.0000x reference)
import jax
import jax.numpy as jnp
from jax.experimental import pallas as pl


def kernel(data, centroids):
    raise NotImplementedError("write your pallas kernel here")


# speedup vs baseline: 11.5936x; 11.5936x over previous
import jax
import jax.numpy as jnp
from jax.experimental import pallas as pl


def _cluster_kernel(data_ref, cent_ref, out_ref):
    data = data_ref[:, :]
    cent = cent_ref[:, :]
    xx = jnp.sum(data * data, axis=1)
    cc = jnp.sum(cent * cent, axis=1)
    g = jax.lax.dot_general(
        cent, data, (((1,), (1,)), ((), ())),
        preferred_element_type=jnp.float32)
    dist = cc[:, None] + xx[None, :] - 2.0 * g
    q = 1.0 / (1.0 + dist)
    q = q * q * 0.5
    s = jnp.sum(q, axis=0)
    out_ref[:, :] = q / s[None, :]


def kernel(data, centroids):
    n, _ = data.shape
    k, _ = centroids.shape
    return pl.pallas_call(
        _cluster_kernel,
        out_shape=jax.ShapeDtypeStruct((k, n), jnp.float32),
    )(data, centroids)
